# Initial kernel scaffold; baseline (speedup 1.0000x reference)
import jax
import jax.numpy as jnp
from jax.experimental import pallas as pl


def kernel(x, W_msg, W_act, W_finish, W_effect, W_phase, W_position, W_number, W_place, W_attrib):
    raise NotImplementedError("write your pallas kernel here")


# speedup vs baseline: 4.6839x; 4.6839x over previous
import functools

import jax
import jax.numpy as jnp
from jax import lax
from jax.experimental import pallas as pl
from jax.experimental.pallas import tpu as pltpu
from jax.experimental.pallas import tpu_sc as plsc

_TABLE_ROWS = (30, 10, 3, 256, 4, 9, 13, 31, 10)
_TABLE_DIMS = (16, 16, 8, 32, 8, 16, 8, 16, 8)
_NT = len(_TABLE_DIMS)

_B, _L = 4096, 200
_N = _B * _L

_INFO = plsc.get_sparse_core_info()
_NC, _NS = _INFO.num_cores, _INFO.num_subcores
_NW = _NC * _NS
_TOK_PER_W = _N // _NW
_C = 512
_N_CHUNKS = _TOK_PER_W // _C


def _make_sc_call():
    mesh = plsc.VectorSubcoreMesh(core_axis_name="c", subcore_axis_name="s")
    out_type = [jax.ShapeDtypeStruct((_N * d,), jnp.float32) for d in _TABLE_DIMS]
    scratch = (
        [pltpu.VMEM((n * d,), jnp.float32) for n, d in zip(_TABLE_ROWS, _TABLE_DIMS)]
        + [pltpu.VMEM((_C * _NT,), jnp.int32)]
        + [pltpu.VMEM((_C * d,), jnp.float32) for d in _TABLE_DIMS]
    )

    @functools.partial(
        pl.kernel,
        out_type=out_type,
        mesh=mesh,
        scratch_types=scratch,
        compiler_params=pltpu.CompilerParams(needs_layout_passes=False),
    )
    def sc_fn(*refs):
        x_hbm = refs[0]
        w_hbm = refs[1 : 1 + _NT]
        outs_hbm = refs[1 + _NT : 1 + 2 * _NT]
        tabs = refs[1 + 2 * _NT : 1 + 3 * _NT]
        xv = refs[1 + 3 * _NT]
        obufs = refs[2 + 3 * _NT : 2 + 4 * _NT]

        wid = lax.axis_index("s") * _NC + lax.axis_index("c")
        base0 = wid * _TOK_PER_W

        for k in range(_NT):
            pltpu.sync_copy(w_hbm[k], tabs[k])

        lanes = lax.iota(jnp.int32, 16)

        def chunk_body(ci, carry):
            base = base0 + ci * _C
            pltpu.sync_copy(x_hbm.at[pl.ds(base * _NT, _C * _NT)], xv)

            def group_body(g, c2):
                tok = g * 16 + lanes
                for k in range(_NT):
                    xk = plsc.load_gather(xv, [tok * _NT + k])
                    d = _TABLE_DIMS[k]
                    xkd = xk * d
                    tokd = tok * d
                    for j in range(d):
                        vals = plsc.load_gather(tabs[k], [xkd + j])
                        plsc.store_scatter(obufs[k], [tokd + j], vals)
                return c2

            lax.fori_loop(0, _C // 16, group_body, 0)

            for k in range(_NT):
                d = _TABLE_DIMS[k]
                pltpu.sync_copy(obufs[k], outs_hbm[k].at[pl.ds(base * d, _C * d)])
            return carry

        lax.fori_loop(0, _N_CHUNKS, chunk_body, 0)

    return sc_fn


_SC_CALL = _make_sc_call()


def kernel(x, W_msg, W_act, W_finish, W_effect, W_phase, W_position, W_number,
           W_place, W_attrib):
    ws = (W_msg, W_act, W_finish, W_effect, W_phase, W_position, W_number,
          W_place, W_attrib)
    outs = _SC_CALL(x.reshape(_N * _NT), *(w.reshape(-1) for w in ws))
    return tuple(o.reshape(_B, _L, d) for o, d in zip(outs, _TABLE_DIMS))


# speedup vs baseline: 7.0553x; 1.5063x over previous
import functools

import jax
import jax.numpy as jnp
from jax import lax
from jax.experimental import pallas as pl
from jax.experimental.pallas import tpu as pltpu
from jax.experimental.pallas import tpu_sc as plsc

_TABLE_ROWS = (30, 10, 3, 256, 4, 9, 13, 31, 10)
_TABLE_DIMS = (16, 16, 8, 32, 8, 16, 8, 16, 8)
_NT = len(_TABLE_DIMS)

_B, _L = 4096, 200
_N = _B * _L

_INFO = plsc.get_sparse_core_info()
_NC, _NS = _INFO.num_cores, _INFO.num_subcores
_NW = _NC * _NS
_TOK_PER_W = _N // _NW
_C = 128
_NCH = _TOK_PER_W // _C


def _vmem_shared(shape, dtype):
    return pltpu.VMEM_SHARED(shape, dtype)


def _make_sc_call():
    mesh = plsc.VectorSubcoreMesh(core_axis_name="c", subcore_axis_name="s")
    out_type = [jax.ShapeDtypeStruct((_N, d), jnp.float32) for d in _TABLE_DIMS]
    scratch = (
        [_vmem_shared((n, d), jnp.float32) for n, d in zip(_TABLE_ROWS, _TABLE_DIMS)]
        + [pltpu.VMEM((_C * _NT,), jnp.int32) for _ in range(2)]
        + [pltpu.VMEM((_C,), jnp.int32) for _ in range(2 * _NT)]
        + [pltpu.VMEM((_C, d), jnp.float32) for _ in range(2) for d in _TABLE_DIMS]
        + [pltpu.SemaphoreType.DMA for _ in range(5)]
    )

    @functools.partial(
        pl.kernel,
        out_type=out_type,
        mesh=mesh,
        scratch_types=scratch,
        compiler_params=pltpu.CompilerParams(
            needs_layout_passes=False, use_tc_tiling_on_sc=False
        ),
    )
    def sc_fn(*refs):
        it = iter(refs)
        x_hbm = next(it)
        w_hbm = [next(it) for _ in range(_NT)]
        outs_hbm = [next(it) for _ in range(_NT)]
        tabs = [next(it) for _ in range(_NT)]
        xv = [next(it) for _ in range(2)]
        idxb = [[next(it) for _ in range(_NT)] for _ in range(2)]
        obuf = [[next(it) for _ in range(_NT)] for _ in range(2)]
        xsem = [next(it) for _ in range(2)]
        gsem = next(it)
        osem = [next(it) for _ in range(2)]

        sid = lax.axis_index("s")
        wid = sid * _NC + lax.axis_index("c")
        base0 = wid * _TOK_PER_W

        @pl.when(sid == 0)
        def _():
            for k in range(_NT):
                pltpu.sync_copy(w_hbm[k], tabs[k])

        plsc.subcore_barrier()

        lanes = lax.iota(jnp.int32, 16)

        def x_copy(ci, s):
            return pltpu.make_async_copy(
                x_hbm.at[pl.ds((base0 + ci * _C) * _NT, _C * _NT)], xv[s], xsem[s]
            )

        def out_copy(ci, s, k):
            return pltpu.make_async_copy(
                obuf[s][k], outs_hbm[k].at[pl.ds(base0 + ci * _C, _C)], osem[s]
            )

        x_copy(0, 0).start()

        def process_chunk(ci, s, not_first):
            @pl.when(ci + 1 < _NCH)
            def _():
                x_copy(ci + 1, 1 - s).start()

            x_copy(ci, s).wait()

            for g in range(_C // 16):
                tok9 = (g * 16 + lanes) * _NT
                for k in range(_NT):
                    xk = plsc.load_gather(xv[s], [tok9 + k])
                    idxb[s][k][pl.ds(g * 16, 16)] = xk

            @pl.when(not_first)
            def _():
                for k in range(_NT):
                    out_copy(ci, s, k).wait()

            gathers = [
                pltpu.make_async_copy(tabs[k].at[idxb[s][k]], obuf[s][k], gsem)
                for k in range(_NT)
            ]
            for g_ in gathers:
                g_.start()
            for g_ in gathers:
                g_.wait()

            for k in range(_NT):
                out_copy(ci, s, k).start()

        def pair_body(h, carry):
            process_chunk(2 * h, 0, h >= 1)
            process_chunk(2 * h + 1, 1, h >= 1)
            return carry

        lax.fori_loop(0, _NCH // 2, pair_body, 0)

        for s in range(2):
            for k in range(_NT):
                out_copy(0, s, k).wait()

    return sc_fn


_SC_CALL = _make_sc_call()


def kernel(x, W_msg, W_act, W_finish, W_effect, W_phase, W_position, W_number,
           W_place, W_attrib):
    ws = (W_msg, W_act, W_finish, W_effect, W_phase, W_position, W_number,
          W_place, W_attrib)
    outs = _SC_CALL(x.reshape(_N * _NT), *ws)
    return tuple(o.reshape(_B, _L, d) for o, d in zip(outs, _TABLE_DIMS))
